# baseline (speedup 1.0000x reference)
import functools

import jax
import jax.numpy as jnp
from jax.experimental import pallas as pl


def _matmul_bias_kernel(a_ref, w_ref, b_ref, o_ref):
    o_ref[...] = (
        jnp.dot(a_ref[...], w_ref[...], preferred_element_type=jnp.float32)
        + b_ref[...]
    )


@functools.partial(jax.jit, static_argnames=("block_m",))
def _patch_embed(x, W, b, block_m: int = 784):
    B = x.shape[0]
    a = (
        x.reshape(B, 3, 14, 16, 14, 16)
        .transpose(0, 2, 4, 1, 3, 5)
        .reshape(B * 196, 768)
    )
    w = W.reshape(768, 768).T
    bias = b.reshape(1, 768)
    m = a.shape[0]
    grid = (m // block_m,)
    out = pl.pallas_call(
        _matmul_bias_kernel,
        grid=grid,
        in_specs=[
            pl.BlockSpec((block_m, 768), lambda i: (i, 0)),
            pl.BlockSpec((768, 768), lambda i: (0, 0)),
            pl.BlockSpec((1, 768), lambda i: (0, 0)),
        ],
        out_specs=pl.BlockSpec((block_m, 768), lambda i: (i, 0)),
        out_shape=jax.ShapeDtypeStruct((m, 768), jnp.float32),
    )(a, w, bias)
    return out.reshape(B, 196, 768)


def kernel(x, W, b):
    return _patch_embed(x, W, b)


# speedup vs baseline: 2.2691x; 2.2691x over previous
import functools

import jax
import jax.numpy as jnp
from jax.experimental import pallas as pl


def _fused_kernel(x_ref, w_ref, b_ref, o_ref):
    v = x_ref[0].reshape(3, 14, 16, 14, 16)
    a = jnp.transpose(v, (1, 3, 0, 2, 4)).reshape(196, 768)
    o_ref[0] = (
        jnp.dot(a, w_ref[...], preferred_element_type=jnp.float32) + b_ref[...]
    )


@jax.jit
def _patch_embed(x, W, b):
    B = x.shape[0]
    w = W.reshape(768, 768).T
    bias = b.reshape(1, 768)
    out = pl.pallas_call(
        _fused_kernel,
        grid=(B,),
        in_specs=[
            pl.BlockSpec((1, 3, 224, 224), lambda i: (i, 0, 0, 0)),
            pl.BlockSpec((768, 768), lambda i: (0, 0)),
            pl.BlockSpec((1, 768), lambda i: (0, 0)),
        ],
        out_specs=pl.BlockSpec((1, 196, 768), lambda i: (i, 0, 0)),
        out_shape=jax.ShapeDtypeStruct((B, 196, 768), jnp.float32),
    )(x, w, bias)
    return out


def kernel(x, W, b):
    return _patch_embed(x, W, b)


# speedup vs baseline: 2.2715x; 1.0011x over previous
import functools

import jax
import jax.numpy as jnp
from jax.experimental import pallas as pl
from jax.experimental.pallas import tpu as pltpu


def _fused_kernel(x_ref, w_ref, b_ref, o_ref):
    v = x_ref[0].reshape(3, 14, 16, 14, 16)
    a = jnp.transpose(v, (1, 3, 0, 2, 4)).reshape(196, 768)
    o_ref[0] = (
        jnp.dot(a, w_ref[...], preferred_element_type=jnp.float32) + b_ref[...]
    )


@jax.jit
def _patch_embed(x, W, b):
    B = x.shape[0]
    w = W.reshape(768, 768).T
    bias = b.reshape(1, 768)
    out = pl.pallas_call(
        _fused_kernel,
        grid=(B,),
        in_specs=[
            pl.BlockSpec((1, 3, 224, 224), lambda i: (i, 0, 0, 0)),
            pl.BlockSpec((768, 768), lambda i: (0, 0)),
            pl.BlockSpec((1, 768), lambda i: (0, 0)),
        ],
        out_specs=pl.BlockSpec((1, 196, 768), lambda i: (i, 0, 0)),
        out_shape=jax.ShapeDtypeStruct((B, 196, 768), jnp.float32),
        compiler_params=pltpu.CompilerParams(
            dimension_semantics=("parallel",)
        ),
    )(x, w, bias)
    return out


def kernel(x, W, b):
    return _patch_embed(x, W, b)


# speedup vs baseline: 2.9002x; 1.2768x over previous
import functools

import jax
import jax.numpy as jnp
from jax.experimental import pallas as pl
from jax.experimental.pallas import tpu as pltpu


def _fused_kernel(x_ref, w_ref, b_ref, o_ref):
    v = x_ref[0].astype(jnp.bfloat16).reshape(3, 14, 16, 14, 16)
    a = jnp.transpose(v, (1, 3, 0, 2, 4)).reshape(196, 768)
    o_ref[0] = (
        jnp.dot(a, w_ref[...], preferred_element_type=jnp.float32) + b_ref[...]
    )


@jax.jit
def _patch_embed(x, W, b):
    B = x.shape[0]
    w = W.reshape(768, 768).T.astype(jnp.bfloat16)
    bias = b.reshape(1, 768)
    out = pl.pallas_call(
        _fused_kernel,
        grid=(B,),
        in_specs=[
            pl.BlockSpec((1, 3, 224, 224), lambda i: (i, 0, 0, 0)),
            pl.BlockSpec((768, 768), lambda i: (0, 0)),
            pl.BlockSpec((1, 768), lambda i: (0, 0)),
        ],
        out_specs=pl.BlockSpec((1, 196, 768), lambda i: (i, 0, 0)),
        out_shape=jax.ShapeDtypeStruct((B, 196, 768), jnp.float32),
        compiler_params=pltpu.CompilerParams(
            dimension_semantics=("parallel",)
        ),
    )(x, w, bias)
    return out


def kernel(x, W, b):
    return _patch_embed(x, W, b)


# speedup vs baseline: 3.6693x; 1.2652x over previous
import functools

import jax
import jax.numpy as jnp
from jax.experimental import pallas as pl
from jax.experimental.pallas import tpu as pltpu


def _fused_kernel(x_ref, w_ref, b_ref, o_ref):
    u = x_ref[0].astype(jnp.bfloat16).reshape(3, 14, 16, 224)
    u = jnp.transpose(u, (1, 0, 2, 3)).reshape(14, 48, 224)
    t = jnp.transpose(u, (0, 2, 1)).reshape(14, 14, 16, 48)
    acc = jnp.zeros((196, 768), jnp.float32)
    for kw in range(16):
        s = t[:, :, kw, :].reshape(196, 48)
        acc += jnp.dot(s, w_ref[kw], preferred_element_type=jnp.float32)
    o_ref[0] = acc + b_ref[...]


@jax.jit
def _patch_embed(x, W, b):
    B = x.shape[0]
    w = W.transpose(3, 1, 2, 0).reshape(16, 48, 768).astype(jnp.bfloat16)
    bias = b.reshape(1, 768)
    out = pl.pallas_call(
        _fused_kernel,
        grid=(B,),
        in_specs=[
            pl.BlockSpec((1, 3, 224, 224), lambda i: (i, 0, 0, 0)),
            pl.BlockSpec((16, 48, 768), lambda i: (0, 0, 0)),
            pl.BlockSpec((1, 768), lambda i: (0, 0)),
        ],
        out_specs=pl.BlockSpec((1, 196, 768), lambda i: (i, 0, 0)),
        out_shape=jax.ShapeDtypeStruct((B, 196, 768), jnp.float32),
        compiler_params=pltpu.CompilerParams(
            dimension_semantics=("parallel",)
        ),
    )(x, w, bias)
    return out


def kernel(x, W, b):
    return _patch_embed(x, W, b)
